# baseline (speedup 1.0000x reference)
import functools

import jax
import jax.numpy as jnp
from jax import lax
from jax.experimental import pallas as pl
from jax.experimental.pallas import tpu as pltpu
from jax.experimental.pallas import tpu_sc as plsc

B, S, H = 4, 2048, 768
V, P, T, G = 100000, 4096, 2, 50
N = B * S
EPS = 1e-12

L = 16
NCOL = H // L
C = 16

_DNUMS = lax.GatherDimensionNumbers(
    offset_dims=(), collapsed_slice_dims=(0,), start_index_map=(0,))


def _shuffle(x, idx):
    return lax.gather(x, idx[:, None], _DNUMS, slice_sizes=(1,),
                      mode=lax.GatherScatterMode.PROMISE_IN_BOUNDS)


def _splat_lane(v, t):
    return _shuffle(v, jnp.full((L,), t, jnp.int32))


def _rsqrt(v):
    half = v * 0.5
    y = plsc.bitcast(
        jnp.int32(0x5F3759DF) - (plsc.bitcast(v, jnp.int32) >> 1),
        jnp.float32)
    y = y * (1.5 - half * y * y)
    y = y * (1.5 - half * y * y)
    y = y * (1.5 - half * y * y)
    return y


def _build_sc_kernel():
    info = plsc.get_sparse_core_info()
    nc, ns = info.num_cores, info.num_subcores
    nw = nc * ns
    tw = N // nw
    nch = tw // C
    mesh = plsc.VectorSubcoreMesh(core_axis_name="c", subcore_axis_name="s")

    @functools.partial(
        pl.kernel,
        mesh=mesh,
        compiler_params=pltpu.CompilerParams(needs_layout_passes=False),
        out_type=jax.ShapeDtypeStruct((N, H), jnp.float32),
        scratch_types=[
            pltpu.VMEM((tw,), jnp.int32),
            pltpu.VMEM((tw,), jnp.int32),
            pltpu.VMEM((C, H), jnp.float32),
            pltpu.VMEM((C, H), jnp.float32),
            pltpu.VMEM((C, H // 2), jnp.int32),
            pltpu.VMEM((C, H // 2), jnp.int32),
            pltpu.VMEM((C, H), jnp.float32),
            pltpu.VMEM((C, H), jnp.float32),
            pltpu.VMEM((T * G, H // 2), jnp.int32),
            pltpu.VMEM((H,), jnp.float32),
            pltpu.VMEM((H,), jnp.float32),
            pltpu.VMEM((C, 33), jnp.float32),
            pltpu.SemaphoreType.DMA,
            pltpu.SemaphoreType.DMA,
            pltpu.SemaphoreType.DMA,
            pltpu.SemaphoreType.DMA,
        ],
    )
    def emb_ln(ids_h, cid_h, word_h, pos_h, combo_h, gam_h, bet_h, out_h,
               ids_v, cid_v, wbuf0, wbuf1, pbuf0, pbuf1, obuf0, obuf1,
               combo_v, gam_v, bet_v, statb, gsem0, gsem1, ssem0, ssem1):
        wid = lax.axis_index("s") * nc + lax.axis_index("c")
        tbase = wid * tw
        pltpu.sync_copy(ids_h.at[pl.ds(tbase, tw)], ids_v)
        pltpu.sync_copy(cid_h.at[pl.ds(tbase, tw)], cid_v)
        pltpu.sync_copy(combo_h, combo_v)
        pltpu.sync_copy(gam_h, gam_v)
        pltpu.sync_copy(bet_h, bet_v)

        def issue(g, wbuf, pbuf, gsem):
            ivec = ids_v[pl.ds(g * C, C)]
            pltpu.async_copy(word_h.at[ivec], wbuf, gsem)
            sb = lax.rem(tbase + g * C, S)
            pltpu.async_copy(pos_h.at[pl.ds(sb, C)], pbuf, gsem)

        issue(0, wbuf0, pbuf0, gsem0)
        issue(1, wbuf1, pbuf1, gsem1)

        def compute(g, wbuf, pbuf, obuf):
            blocal = g * C
            ctv = cid_v[pl.ds(blocal, C)]
            lane = lax.iota(jnp.int32, L)

            zero = jnp.zeros((L,), jnp.float32)
            for t in range(C):
                cr = ctv[t]

                def col_sum(j, carry, t=t, cr=cr):
                    a1, a2, b1, b2 = carry
                    c0 = pl.ds(j * 2 * L, L)
                    c1 = pl.ds((j * 2 + 1) * L, L)
                    raw = plsc.bitcast(combo_v[cr, pl.ds(j * L, L)],
                                       jnp.bfloat16)
                    ca, cb = plsc.unpack(raw,
                                         format=plsc.PackFormat.INTERLEAVED)
                    rawp = plsc.bitcast(pbuf[t, pl.ds(j * L, L)],
                                        jnp.bfloat16)
                    pa, pb = plsc.unpack(rawp,
                                         format=plsc.PackFormat.INTERLEAVED)
                    x0 = wbuf[t, c0] + pa + ca
                    x1 = wbuf[t, c1] + pb + cb
                    obuf[t, c0] = x0
                    obuf[t, c1] = x1
                    return (a1 + x0, a2 + x0 * x0, b1 + x1, b2 + x1 * x1)

                a1, a2, b1, b2 = lax.fori_loop(
                    0, NCOL // 2, col_sum, (zero, zero, zero, zero),
                    unroll=4)
                statb[t, pl.ds(0, L)] = a1 + b1
                statb[t, pl.ds(16, L)] = a2 + b2

            t1 = plsc.load_gather(statb, [lane, jnp.full((L,), 0, jnp.int32)])
            t2 = plsc.load_gather(statb, [lane, jnp.full((L,), 16, jnp.int32)])
            for cc in range(1, L):
                t1 = t1 + plsc.load_gather(
                    statb, [lane, jnp.full((L,), cc, jnp.int32)])
                t2 = t2 + plsc.load_gather(
                    statb, [lane, jnp.full((L,), 16 + cc, jnp.int32)])
            mu = t1 * (1.0 / H)
            rho = _rsqrt(t2 * (1.0 / H) - mu * mu + EPS)
            mus = [_splat_lane(mu, t) for t in range(C)]
            rhos = [_splat_lane(rho, t) for t in range(C)]

            def col_norm(j, _):
                cj = pl.ds(j * L, L)
                gj = gam_v[cj]
                bj = bet_v[cj]
                for t in range(C):
                    obuf[t, cj] = (obuf[t, cj] - mus[t]) * rhos[t] * gj + bj
                return 0

            lax.fori_loop(0, NCOL, col_norm, 0)

        def step(g, wbuf, pbuf, obuf, gsem, ssem):
            pltpu.make_async_copy(word_h.at[pl.ds(0, C)], wbuf, gsem).wait()
            pltpu.make_async_copy(pos_h.at[pl.ds(0, C)], pbuf, gsem).wait()

            @pl.when(g >= 2)
            def _():
                pltpu.make_async_copy(obuf, out_h.at[pl.ds(0, C)],
                                      ssem).wait()

            compute(g, wbuf, pbuf, obuf)

            @pl.when(g + 2 < nch)
            def _():
                issue(g + 2, wbuf, pbuf, gsem)

            pltpu.async_copy(obuf, out_h.at[pl.ds(tbase + g * C, C)], ssem)

        def pair(i2, _):
            step(i2 * 2, wbuf0, pbuf0, obuf0, gsem0, ssem0)
            step(i2 * 2 + 1, wbuf1, pbuf1, obuf1, gsem1, ssem1)
            return 0

        lax.fori_loop(0, nch // 2, pair, 0)
        pltpu.make_async_copy(obuf0, out_h.at[pl.ds(0, C)], ssem0).wait()
        pltpu.make_async_copy(obuf1, out_h.at[pl.ds(0, C)], ssem1).wait()

    return emb_ln


def kernel(input_ids, token_type_ids, part_of_speech_ids, word_emb,
           position_emb, token_type_emb, pos_tag_emb, gamma, beta):
    ids = input_ids.reshape(-1).astype(jnp.int32)
    cidx = (token_type_ids.reshape(-1).astype(jnp.int32) * G
            + part_of_speech_ids.reshape(-1).astype(jnp.int32))
    combo = (token_type_emb[:, None, :]
             + pos_tag_emb[None, :, :]).reshape(T * G, H)
    combo = (combo.reshape(T * G, H // 32, 2, 16)
             .swapaxes(2, 3).astype(jnp.bfloat16))
    combo = lax.bitcast_convert_type(combo, jnp.int32).reshape(T * G, H // 2)
    posp = (position_emb.reshape(P, H // 32, 2, 16)
            .swapaxes(2, 3).astype(jnp.bfloat16))
    posp = lax.bitcast_convert_type(posp, jnp.int32).reshape(P, H // 2)
    emb_ln = _build_sc_kernel()
    out = emb_ln(ids, cidx, word_emb, posp, combo, gamma, beta)
    return out.reshape(B, S, H)


# speedup vs baseline: 1.1704x; 1.1704x over previous
import functools

import jax
import jax.numpy as jnp
from jax import lax
from jax.experimental import pallas as pl
from jax.experimental.pallas import tpu as pltpu
from jax.experimental.pallas import tpu_sc as plsc

B, S, H = 4, 2048, 768
V, P, T, G = 100000, 4096, 2, 50
N = B * S
EPS = 1e-12

L = 16
NCOL = H // L
C = 16

_DNUMS = lax.GatherDimensionNumbers(
    offset_dims=(), collapsed_slice_dims=(0,), start_index_map=(0,))


def _shuffle(x, idx):
    return lax.gather(x, idx[:, None], _DNUMS, slice_sizes=(1,),
                      mode=lax.GatherScatterMode.PROMISE_IN_BOUNDS)


def _splat_lane(v, t):
    return _shuffle(v, jnp.full((L,), t, jnp.int32))


def _rsqrt(v):
    half = v * 0.5
    y = plsc.bitcast(
        jnp.int32(0x5F3759DF) - (plsc.bitcast(v, jnp.int32) >> 1),
        jnp.float32)
    y = y * (1.5 - half * y * y)
    y = y * (1.5 - half * y * y)
    y = y * (1.5 - half * y * y)
    return y


def _build_sc_kernel():
    info = plsc.get_sparse_core_info()
    nc, ns = info.num_cores, info.num_subcores
    nw = nc * ns
    tw = N // nw
    nch = tw // C
    mesh = plsc.VectorSubcoreMesh(core_axis_name="c", subcore_axis_name="s")

    @functools.partial(
        pl.kernel,
        mesh=mesh,
        compiler_params=pltpu.CompilerParams(needs_layout_passes=False),
        out_type=jax.ShapeDtypeStruct((N, H), jnp.float32),
        scratch_types=[
            pltpu.VMEM((tw,), jnp.int32),
            pltpu.VMEM((tw,), jnp.int32),
            pltpu.VMEM((C, H), jnp.float32),
            pltpu.VMEM((C, H), jnp.float32),
            pltpu.VMEM((C, H), jnp.float32),
            pltpu.VMEM((C, H), jnp.float32),
            pltpu.VMEM((C, H), jnp.float32),
            pltpu.VMEM((C, H), jnp.float32),
            pltpu.VMEM((T * G, H // 2), jnp.int32),
            pltpu.VMEM((H,), jnp.float32),
            pltpu.VMEM((H,), jnp.float32),
            pltpu.VMEM((C, 33), jnp.float32),
            pltpu.SemaphoreType.DMA,
            pltpu.SemaphoreType.DMA,
            pltpu.SemaphoreType.DMA,
            pltpu.SemaphoreType.DMA,
        ],
    )
    def emb_ln(ids_h, cid_h, word_h, pos_h, combo_h, gam_h, bet_h, out_h,
               ids_v, cid_v, wbuf0, wbuf1, pbuf0, pbuf1, obuf0, obuf1,
               combo_v, gam_v, bet_v, statb, gsem0, gsem1, ssem0, ssem1):
        wid = lax.axis_index("s") * nc + lax.axis_index("c")
        tbase = wid * tw
        pltpu.sync_copy(ids_h.at[pl.ds(tbase, tw)], ids_v)
        pltpu.sync_copy(cid_h.at[pl.ds(tbase, tw)], cid_v)
        pltpu.sync_copy(combo_h, combo_v)
        pltpu.sync_copy(gam_h, gam_v)
        pltpu.sync_copy(bet_h, bet_v)

        def issue(g, wbuf, pbuf, gsem):
            ivec = ids_v[pl.ds(g * C, C)]
            pltpu.async_copy(word_h.at[ivec], wbuf, gsem)
            sb = lax.rem(tbase + g * C, S)
            pltpu.async_copy(pos_h.at[pl.ds(sb, C)], pbuf, gsem)

        issue(0, wbuf0, pbuf0, gsem0)
        issue(1, wbuf1, pbuf1, gsem1)

        def compute(g, wbuf, pbuf, obuf):
            blocal = g * C
            ctv = cid_v[pl.ds(blocal, C)]
            lane = lax.iota(jnp.int32, L)

            zero = jnp.zeros((L,), jnp.float32)
            for t in range(C):
                cr = ctv[t]

                def col_sum(j, carry, t=t, cr=cr):
                    a1, a2, b1, b2 = carry
                    c0 = pl.ds(j * 2 * L, L)
                    c1 = pl.ds((j * 2 + 1) * L, L)
                    raw = plsc.bitcast(combo_v[cr, pl.ds(j * L, L)],
                                       jnp.bfloat16)
                    ca, cb = plsc.unpack(raw,
                                         format=plsc.PackFormat.INTERLEAVED)
                    x0 = wbuf[t, c0] + pbuf[t, c0] + ca
                    x1 = wbuf[t, c1] + pbuf[t, c1] + cb
                    obuf[t, c0] = x0
                    obuf[t, c1] = x1
                    return (a1 + x0, a2 + x0 * x0, b1 + x1, b2 + x1 * x1)

                a1, a2, b1, b2 = lax.fori_loop(
                    0, NCOL // 2, col_sum, (zero, zero, zero, zero),
                    unroll=6)
                statb[t, pl.ds(0, L)] = a1 + b1
                statb[t, pl.ds(16, L)] = a2 + b2

            t1 = plsc.load_gather(statb, [lane, jnp.full((L,), 0, jnp.int32)])
            t2 = plsc.load_gather(statb, [lane, jnp.full((L,), 16, jnp.int32)])
            for cc in range(1, L):
                t1 = t1 + plsc.load_gather(
                    statb, [lane, jnp.full((L,), cc, jnp.int32)])
                t2 = t2 + plsc.load_gather(
                    statb, [lane, jnp.full((L,), 16 + cc, jnp.int32)])
            mu = t1 * (1.0 / H)
            rho = _rsqrt(t2 * (1.0 / H) - mu * mu + EPS)
            mus = [_splat_lane(mu, t) for t in range(C)]
            rhos = [_splat_lane(rho, t) for t in range(C)]

            def col_norm(j, _):
                cj = pl.ds(j * L, L)
                gj = gam_v[cj]
                bj = bet_v[cj]
                for t in range(C):
                    obuf[t, cj] = (obuf[t, cj] - mus[t]) * rhos[t] * gj + bj
                return 0

            lax.fori_loop(0, NCOL, col_norm, 0)

        def step(g, wbuf, pbuf, obuf, gsem, ssem):
            pltpu.make_async_copy(word_h.at[pl.ds(0, C)], wbuf, gsem).wait()
            pltpu.make_async_copy(pos_h.at[pl.ds(0, C)], pbuf, gsem).wait()

            @pl.when(g >= 2)
            def _():
                pltpu.make_async_copy(obuf, out_h.at[pl.ds(0, C)],
                                      ssem).wait()

            compute(g, wbuf, pbuf, obuf)

            @pl.when(g + 2 < nch)
            def _():
                issue(g + 2, wbuf, pbuf, gsem)

            pltpu.async_copy(obuf, out_h.at[pl.ds(tbase + g * C, C)], ssem)

        def pair(i2, _):
            step(i2 * 2, wbuf0, pbuf0, obuf0, gsem0, ssem0)
            step(i2 * 2 + 1, wbuf1, pbuf1, obuf1, gsem1, ssem1)
            return 0

        lax.fori_loop(0, nch // 2, pair, 0)
        pltpu.make_async_copy(obuf0, out_h.at[pl.ds(0, C)], ssem0).wait()
        pltpu.make_async_copy(obuf1, out_h.at[pl.ds(0, C)], ssem1).wait()

    return emb_ln


def kernel(input_ids, token_type_ids, part_of_speech_ids, word_emb,
           position_emb, token_type_emb, pos_tag_emb, gamma, beta):
    ids = input_ids.reshape(-1).astype(jnp.int32)
    cidx = (token_type_ids.reshape(-1).astype(jnp.int32) * G
            + part_of_speech_ids.reshape(-1).astype(jnp.int32))
    combo = (token_type_emb[:, None, :]
             + pos_tag_emb[None, :, :]).reshape(T * G, H)
    combo = (combo.reshape(T * G, H // 32, 2, 16)
             .swapaxes(2, 3).astype(jnp.bfloat16))
    combo = lax.bitcast_convert_type(combo, jnp.int32).reshape(T * G, H // 2)
    emb_ln = _build_sc_kernel()
    out = emb_ln(ids, cidx, word_emb, position_emb, combo, gamma, beta)
    return out.reshape(B, S, H)


# speedup vs baseline: 1.3505x; 1.1539x over previous
import functools

import jax
import jax.numpy as jnp
from jax import lax
from jax.experimental import pallas as pl
from jax.experimental.pallas import tpu as pltpu
from jax.experimental.pallas import tpu_sc as plsc

B, S, H = 4, 2048, 768
V, P, T, G = 100000, 4096, 2, 50
N = B * S
EPS = 1e-12

L = 16
NCOL = H // L
C = 16

_DNUMS = lax.GatherDimensionNumbers(
    offset_dims=(), collapsed_slice_dims=(0,), start_index_map=(0,))


def _shuffle(x, idx):
    return lax.gather(x, idx[:, None], _DNUMS, slice_sizes=(1,),
                      mode=lax.GatherScatterMode.PROMISE_IN_BOUNDS)


def _splat_lane(v, t):
    return _shuffle(v, jnp.full((L,), t, jnp.int32))


def _rsqrt(v):
    half = v * 0.5
    y = plsc.bitcast(
        jnp.int32(0x5F3759DF) - (plsc.bitcast(v, jnp.int32) >> 1),
        jnp.float32)
    y = y * (1.5 - half * y * y)
    y = y * (1.5 - half * y * y)
    y = y * (1.5 - half * y * y)
    return y


def _build_sc_kernel():
    info = plsc.get_sparse_core_info()
    nc, ns = info.num_cores, info.num_subcores
    nw = nc * ns
    tw = N // nw
    nch = tw // C
    mesh = plsc.VectorSubcoreMesh(core_axis_name="c", subcore_axis_name="s")

    @functools.partial(
        pl.kernel,
        mesh=mesh,
        compiler_params=pltpu.CompilerParams(needs_layout_passes=False),
        out_type=jax.ShapeDtypeStruct((N, H), jnp.float32),
        scratch_types=[
            pltpu.VMEM((tw,), jnp.int32),
            pltpu.VMEM((tw,), jnp.int32),
            pltpu.VMEM((C, H), jnp.float32),
            pltpu.VMEM((C, H), jnp.float32),
            pltpu.VMEM((C, H), jnp.float32),
            pltpu.VMEM((C, H), jnp.float32),
            pltpu.VMEM((C, H), jnp.float32),
            pltpu.VMEM((C, H), jnp.float32),
            pltpu.VMEM((T * G, H // 2), jnp.int32),
            pltpu.VMEM((H,), jnp.float32),
            pltpu.VMEM((H,), jnp.float32),
            pltpu.VMEM((C, 33), jnp.float32),
            pltpu.SemaphoreType.DMA,
            pltpu.SemaphoreType.DMA,
            pltpu.SemaphoreType.DMA,
            pltpu.SemaphoreType.DMA,
        ],
    )
    def emb_ln(ids_h, cid_h, word_h, pos_h, combo_h, gam_h, bet_h, out_h,
               ids_v, cid_v, wbuf0, wbuf1, pbuf0, pbuf1, obuf0, obuf1,
               combo_v, gam_v, bet_v, statb, gsem0, gsem1, ssem0, ssem1):
        wid = lax.axis_index("s") * nc + lax.axis_index("c")
        tbase = wid * tw
        pltpu.sync_copy(ids_h.at[pl.ds(tbase, tw)], ids_v)
        pltpu.sync_copy(cid_h.at[pl.ds(tbase, tw)], cid_v)
        pltpu.sync_copy(combo_h, combo_v)
        pltpu.sync_copy(gam_h, gam_v)
        pltpu.sync_copy(bet_h, bet_v)

        def issue(g, wbuf, pbuf, gsem):
            ivec = ids_v[pl.ds(g * C, C)]
            pltpu.async_copy(word_h.at[ivec], wbuf, gsem)
            sb = lax.rem(tbase + g * C, S)
            pltpu.async_copy(pos_h.at[pl.ds(sb, C)], pbuf, gsem)

        issue(0, wbuf0, pbuf0, gsem0)
        issue(1, wbuf1, pbuf1, gsem1)

        def compute(g, wbuf, pbuf, obuf):
            blocal = g * C
            ctv = cid_v[pl.ds(blocal, C)]
            lane = lax.iota(jnp.int32, L)

            zero = jnp.zeros((L,), jnp.float32)
            for t in range(C):
                cr = ctv[t]

                def col_sum(j, carry, t=t, cr=cr):
                    a1, a2, b1, b2 = carry
                    c0 = pl.ds(j * 2 * L, L)
                    c1 = pl.ds((j * 2 + 1) * L, L)
                    raw = plsc.bitcast(combo_v[cr, pl.ds(j * L, L)],
                                       jnp.bfloat16)
                    ca, cb = plsc.unpack(raw,
                                         format=plsc.PackFormat.INTERLEAVED)
                    x0 = wbuf[t, c0] + pbuf[t, c0] + ca
                    x1 = wbuf[t, c1] + pbuf[t, c1] + cb
                    obuf[t, c0] = x0
                    obuf[t, c1] = x1
                    return (a1 + x0, a2 + x0 * x0, b1 + x1, b2 + x1 * x1)

                a1, a2, b1, b2 = lax.fori_loop(
                    0, NCOL // 2, col_sum, (zero, zero, zero, zero),
                    unroll=4)
                statb[t, pl.ds(0, L)] = a1 + b1
                statb[t, pl.ds(16, L)] = a2 + b2

            t1 = plsc.load_gather(statb, [lane, jnp.full((L,), 0, jnp.int32)])
            t2 = plsc.load_gather(statb, [lane, jnp.full((L,), 16, jnp.int32)])
            for cc in range(1, L):
                t1 = t1 + plsc.load_gather(
                    statb, [lane, jnp.full((L,), cc, jnp.int32)])
                t2 = t2 + plsc.load_gather(
                    statb, [lane, jnp.full((L,), 16 + cc, jnp.int32)])
            mu = t1 * (1.0 / H)
            rho = _rsqrt(t2 * (1.0 / H) - mu * mu + EPS)
            for t0 in (0, C // 2):
                mus = [_splat_lane(mu, t) for t in range(t0, t0 + C // 2)]
                rhos = [_splat_lane(rho, t) for t in range(t0, t0 + C // 2)]

                def col_norm(j, _, t0=t0, mus=mus, rhos=rhos):
                    cj = pl.ds(j * L, L)
                    gj = gam_v[cj]
                    bj = bet_v[cj]
                    for k in range(C // 2):
                        obuf[t0 + k, cj] = ((obuf[t0 + k, cj] - mus[k])
                                            * rhos[k] * gj + bj)
                    return 0

                lax.fori_loop(0, NCOL, col_norm, 0)

        def step(g, wbuf, pbuf, obuf, gsem, ssem):
            pltpu.make_async_copy(word_h.at[pl.ds(0, C)], wbuf, gsem).wait()
            pltpu.make_async_copy(pos_h.at[pl.ds(0, C)], pbuf, gsem).wait()

            @pl.when(g >= 2)
            def _():
                pltpu.make_async_copy(obuf, out_h.at[pl.ds(0, C)],
                                      ssem).wait()

            compute(g, wbuf, pbuf, obuf)

            @pl.when(g + 2 < nch)
            def _():
                issue(g + 2, wbuf, pbuf, gsem)

            pltpu.async_copy(obuf, out_h.at[pl.ds(tbase + g * C, C)], ssem)

        def pair(i2, _):
            step(i2 * 2, wbuf0, pbuf0, obuf0, gsem0, ssem0)
            step(i2 * 2 + 1, wbuf1, pbuf1, obuf1, gsem1, ssem1)
            return 0

        lax.fori_loop(0, nch // 2, pair, 0)
        pltpu.make_async_copy(obuf0, out_h.at[pl.ds(0, C)], ssem0).wait()
        pltpu.make_async_copy(obuf1, out_h.at[pl.ds(0, C)], ssem1).wait()

    return emb_ln


def kernel(input_ids, token_type_ids, part_of_speech_ids, word_emb,
           position_emb, token_type_emb, pos_tag_emb, gamma, beta):
    ids = input_ids.reshape(-1).astype(jnp.int32)
    cidx = (token_type_ids.reshape(-1).astype(jnp.int32) * G
            + part_of_speech_ids.reshape(-1).astype(jnp.int32))
    combo = (token_type_emb[:, None, :]
             + pos_tag_emb[None, :, :]).reshape(T * G, H)
    combo = (combo.reshape(T * G, H // 32, 2, 16)
             .swapaxes(2, 3).astype(jnp.bfloat16))
    combo = lax.bitcast_convert_type(combo, jnp.int32).reshape(T * G, H // 2)
    emb_ln = _build_sc_kernel()
    out = emb_ln(ids, cidx, word_emb, position_emb, combo, gamma, beta)
    return out.reshape(B, S, H)
